# baseline (speedup 1.0000x reference)
import functools

import jax
import jax.numpy as jnp
from jax import lax
from jax.experimental import pallas as pl
from jax.experimental.pallas import tpu as pltpu
from jax.experimental.pallas import tpu_sc as plsc

_N_NODES = 100
_N_STATES = 1000
_BATCH = 16384
_TSTRIDE = 1024
_TWORDS = _N_NODES * _TSTRIDE

_NW = 32
_SPW = _BATCH // _NW
_HC = 128
_NH = _SPW // _HC
_GRP = _HC // 16


def _prep_body(l_ref, tab_ref):
    l = l_ref[...]
    m = jnp.max(l, axis=1, keepdims=True)
    s = jnp.sum(jnp.exp(l - m), axis=1, keepdims=True)
    lse = jnp.log(s) + m
    t = (l - lse) * jnp.float32(1.0 / _N_NODES)
    tp = jnp.concatenate(
        [t, jnp.zeros((_N_NODES, _TSTRIDE - _N_STATES), jnp.float32)], axis=1)
    tab_ref[...] = tp.reshape(_TWORDS // 128, 128)


def _prep_table(logits):
    return pl.pallas_call(
        _prep_body,
        out_shape=jax.ShapeDtypeStruct((_TWORDS // 128, 128), jnp.float32),
    )(logits)


def _sc_gather_sum(xt, tab):
    mesh = plsc.VectorSubcoreMesh(core_axis_name="c", subcore_axis_name="s")

    @functools.partial(
        pl.kernel,
        mesh=mesh,
        out_type=jax.ShapeDtypeStruct((_BATCH,), jnp.float32),
        compiler_params=pltpu.CompilerParams(needs_layout_passes=False),
        scratch_types=[
            pltpu.VMEM((_TWORDS,), jnp.float32),
            pltpu.VMEM((_N_NODES, _HC), jnp.int32),
            pltpu.VMEM((_N_NODES, _HC), jnp.int32),
            pltpu.VMEM((_HC,), jnp.float32),
            pltpu.SemaphoreType.DMA,
            pltpu.SemaphoreType.DMA,
            pltpu.SemaphoreType.DMA,
            pltpu.SemaphoreType.DMA,
        ],
    )
    def k(xt_hbm, tab_hbm, out_hbm,
          tab_v, xa_v, xb_v, out_v,
          sem_t, sem_xa, sem_xb, sem_o):
        wid = lax.axis_index("s") * 2 + lax.axis_index("c")
        base = wid * _SPW

        h_t = pltpu.async_copy(tab_hbm, tab_v, sem_t)
        pltpu.async_copy(xt_hbm.at[:, pl.ds(base, _HC)], xa_v, sem_xa)
        pltpu.async_copy(xt_hbm.at[:, pl.ds(base + _HC, _HC)], xb_v, sem_xb)
        h_t.wait()
        zero = jnp.zeros((16,), jnp.float32)

        def compute_store(hc, xv):
            def body(n, accs, xv=xv):
                noff = n * _TSTRIDE
                new = []
                for g in range(_GRP):
                    xrow = xv[n, pl.ds(g * 16, 16)]
                    val = plsc.load_gather(tab_v, [xrow + noff])
                    new.append(accs[g] + val)
                return tuple(new)

            accs = lax.fori_loop(0, _N_NODES, body,
                                 (zero,) * _GRP, unroll=4)
            for g in range(_GRP):
                out_v[pl.ds(g * 16, 16)] = accs[g]
            pltpu.async_copy(
                out_v, out_hbm.at[pl.ds(base + hc * _HC, _HC)], sem_o).wait()

        def super_body(i, carry):
            pltpu.make_async_copy(
                xt_hbm.at[:, pl.ds(base, _HC)], xa_v, sem_xa).wait()
            compute_store(2 * i, xa_v)

            @pl.when(2 * i + 2 < _NH)
            def _():
                pltpu.async_copy(
                    xt_hbm.at[:, pl.ds(base + (2 * i + 2) * _HC, _HC)],
                    xa_v, sem_xa)

            pltpu.make_async_copy(
                xt_hbm.at[:, pl.ds(base, _HC)], xb_v, sem_xb).wait()
            compute_store(2 * i + 1, xb_v)

            @pl.when(2 * i + 3 < _NH)
            def _():
                pltpu.async_copy(
                    xt_hbm.at[:, pl.ds(base + (2 * i + 3) * _HC, _HC)],
                    xb_v, sem_xb)

            return carry

        lax.fori_loop(0, _NH // 2, super_body, 0)

    return k(xt, tab)


def kernel(x, logits):
    tab = _prep_table(logits)
    return _sc_gather_sum(x.T, tab.reshape(-1))
